# baseline (speedup 1.0000x reference)
import jax
import jax.numpy as jnp
from jax.experimental import pallas as pl

_ANCHORS = ((1.08, 1.19), (3.42, 4.41), (6.63, 11.38), (9.42, 5.11), (16.62, 10.52))
_NA = 5
_NC = 7
_THR = 0.6
_H = 48
_W = 48
_TPAD = 56
_CHUNK = 128
_NCHUNK = (_H * _W) // _CHUNK
_EPS = 1e-12


def _region_loss_kernel(x_ref, t_ref, out_ref):
    t = t_ref[0]
    lab = t[:, 0:1]
    gx = t[:, 1:2] * float(_W)
    gy = t[:, 2:3] * float(_H)
    gw = t[:, 3:4] * float(_W)
    gl = t[:, 4:5] * float(_H)
    gim = t[:, 5:6]
    gre = t[:, 6:7]
    valid = t[:, 1:2] > 0.0
    validf = jnp.where(valid, 1.0, 0.0)
    gif = jnp.clip(jnp.floor(gx), 0.0, float(_W - 1))
    gjf = jnp.clip(jnp.floor(gy), 0.0, float(_H - 1))
    txv = gx - gif
    tyv = gy - gjf
    area_g = gw * gl

    best_v = jnp.full_like(gx, -1.0)
    best_n = jnp.zeros_like(gx)
    best_w = jnp.full_like(gx, _ANCHORS[0][0])
    best_h = jnp.full_like(gx, _ANCHORS[0][1])
    anch_iou = []
    for a, (aw, ah) in enumerate(_ANCHORS):
        inter = jnp.minimum(gw, aw) * jnp.minimum(gl, ah)
        iou = inter / (area_g + aw * ah - inter + 1e-16)
        anch_iou.append(iou)
        upd = iou > best_v
        best_v = jnp.where(upd, iou, best_v)
        best_n = jnp.where(upd, float(a), best_n)
        best_w = jnp.where(upd, aw, best_w)
        best_h = jnp.where(upd, ah, best_h)
    twv = jnp.log(gw / best_w + 1e-16)
    tlv = jnp.log(gl / best_h + 1e-16)

    labcl = jnp.clip(lab, 0.0, float(_NC - 1))
    labcs = [labcl == float(c) for c in range(_NC)]

    hw = gw * 0.5
    hh = gl * 0.5
    b1x1 = gx - hw
    b1x2 = gx + hw
    b1y1 = gy - hh
    b1y2 = gy + hh

    tio = jax.lax.broadcasted_iota(jnp.int32, (_TPAD, 1), 0).astype(jnp.float32)
    lane = jax.lax.broadcasted_iota(jnp.int32, (1, _CHUNK), 1).astype(jnp.float32)

    accE = jnp.zeros((_TPAD, _CHUNK), dtype=jnp.float32)
    acc_cls = jnp.zeros((1, _CHUNK), dtype=jnp.float32)
    acc_conf = jnp.zeros((1, _CHUNK), dtype=jnp.float32)
    acc_ncm = jnp.zeros((1, _CHUNK), dtype=jnp.float32)
    acc_nobj = jnp.zeros((1, _CHUNK), dtype=jnp.float32)
    carry0 = (accE, acc_cls, acc_conf, acc_ncm, acc_nobj)

    for a, (aw, ah) in enumerate(_ANCHORS):
        pa = jnp.logical_and(valid, best_n == float(a))
        zf = jnp.logical_and(anch_iou[a] > _THR, valid)
        base_c = a * (7 + _NC)

        def body(k, carry, base_c=base_c, pa=pa, zf=zf, aw=aw, ah=ah):
            accE, acc_cls, acc_conf, acc_ncm, acc_nobj = carry
            idx = k.astype(jnp.float32) * float(_CHUNK) + lane
            jcell = jnp.floor(idx * (1.0 / float(_W)))
            icell = idx - jcell * float(_W)

            px = jax.nn.sigmoid(x_ref[0, base_c + 0, pl.ds(k, 1), :])
            py = jax.nn.sigmoid(x_ref[0, base_c + 1, pl.ds(k, 1), :])
            pw = x_ref[0, base_c + 2, pl.ds(k, 1), :]
            ph = x_ref[0, base_c + 3, pl.ds(k, 1), :]
            pim = x_ref[0, base_c + 4, pl.ds(k, 1), :]
            pre = x_ref[0, base_c + 5, pl.ds(k, 1), :]
            conf = jax.nn.sigmoid(x_ref[0, base_c + 6, pl.ds(k, 1), :])

            bw = jnp.exp(pw) * aw
            bh = jnp.exp(ph) * ah
            bx = px + icell
            by = py + jcell
            b2x1 = bx - bw * 0.5
            b2x2 = bx + bw * 0.5
            b2y1 = by - bh * 0.5
            b2y2 = by + bh * 0.5
            a2 = bw * bh

            iw = jnp.maximum(jnp.minimum(b1x2, b2x2) - jnp.maximum(b1x1, b2x1), 0.0)
            ih = jnp.maximum(jnp.minimum(b1y2, b2y2) - jnp.maximum(b1y1, b2y1), 0.0)
            inter = iw * ih
            iou = inter / (area_g + a2 - inter + 1e-16) * validf
            cur_iou = jnp.max(iou, axis=0, keepdims=True)

            cellm = jnp.logical_and(gif == icell, gjf == jcell)
            match = jnp.logical_and(cellm, pa)
            zany = jnp.max(jnp.where(jnp.logical_and(cellm, zf), 1.0, 0.0),
                           axis=0, keepdims=True)
            matchf = jnp.where(match, 1.0, 0.0)
            anym = jnp.max(matchf, axis=0, keepdims=True)

            tsel = jnp.where(match, tio, -1.0)
            tmax = jnp.max(tsel, axis=0, keepdims=True)
            w = jnp.where(jnp.logical_and(match, tsel == tmax), 1.0, 0.0)

            d = px - txv
            err = d * d
            d = py - tyv
            err = err + d * d
            d = pw - twv
            err = err + d * d
            d = ph - tlv
            err = err + d * d
            d = pim - gim
            err = err + d * d
            d = pre - gre
            err = err + d * d
            accE = accE + w * err

            logits = [x_ref[0, base_c + 7 + c, pl.ds(k, 1), :] for c in range(_NC)]
            m = logits[0]
            for c in range(1, _NC):
                m = jnp.maximum(m, logits[c])
            s = jnp.exp(logits[0] - m)
            for c in range(1, _NC):
                s = s + jnp.exp(logits[c] - m)
            lse = jnp.log(s) + m
            for c in range(_NC):
                anyc = jnp.max(jnp.where(jnp.logical_and(match, labcs[c]), 1.0, 0.0),
                               axis=0, keepdims=True)
                acc_cls = acc_cls + anyc * (lse - logits[c])

            base = jnp.where(cur_iou > _THR, 0.0, 1.0)
            cm = jnp.where(anym > 0.0, 1.0, jnp.where(zany > 0.0, 0.0, base))
            bce = jnp.where(anym > 0.0, -jnp.log(conf + _EPS),
                            -jnp.log(1.0 - conf + _EPS))
            acc_conf = acc_conf + cm * bce
            acc_ncm = acc_ncm + cm
            acc_nobj = acc_nobj + anym
            return (accE, acc_cls, acc_conf, acc_ncm, acc_nobj)

        carry0 = jax.lax.fori_loop(0, _NCHUNK, body, carry0)

    accE, acc_cls, acc_conf, acc_ncm, acc_nobj = carry0
    acc_obj = jnp.sum(accE, axis=0, keepdims=True)
    out_ref[0] = jnp.concatenate(
        [acc_obj, acc_cls, acc_conf, acc_ncm, acc_nobj], axis=0)


def kernel(x, target):
    nB = x.shape[0]
    nT = target.shape[1]
    xr = x.reshape(nB, _NA * (7 + _NC), _NCHUNK, _CHUNK)
    tp = jnp.pad(target, ((0, 0), (0, _TPAD - nT), (0, 1)))
    out = pl.pallas_call(
        _region_loss_kernel,
        grid=(nB,),
        in_specs=[
            pl.BlockSpec((1, _NA * (7 + _NC), _NCHUNK, _CHUNK),
                         lambda b: (b, 0, 0, 0)),
            pl.BlockSpec((1, _TPAD, 8), lambda b: (b, 0, 0)),
        ],
        out_specs=pl.BlockSpec((1, 5, _CHUNK), lambda b: (b, 0, 0)),
        out_shape=jax.ShapeDtypeStruct((nB, 5, _CHUNK), jnp.float32),
    )(xr, tp)
    sums = jnp.sum(out, axis=(0, 2))
    n_obj = jnp.maximum(sums[4], 1.0)
    n_cm = jnp.maximum(sums[3], 1.0)
    return (sums[0] + sums[1]) / n_obj + sums[2] / n_cm
